# Initial kernel scaffold; baseline (speedup 1.0000x reference)
import jax
import jax.numpy as jnp
from jax.experimental import pallas as pl


def kernel(entity_table, relation_table, action_mask, head, r_space, e_space):
    raise NotImplementedError("write your pallas kernel here")


# speedup vs baseline: 7.5304x; 7.5304x over previous
import functools

import jax
import jax.numpy as jnp
from jax import lax
from jax.experimental import pallas as pl
from jax.experimental.pallas import tpu as pltpu
from jax.experimental.pallas import tpu_sc as plsc

NUM_ENTITIES = 50000
NUM_RELATIONS = 1000
EMBED_DIM = 64
MAX_ACTIONS = 256
BATCH = 1024

NUM_WORKERS = 32
BPW = BATCH // NUM_WORKERS
HALF = 128
PAD_D = 128


def _sc_body(ent_hbm, rel_hbm, mask_hbm, head_hbm, rsp_hbm, esp_hbm, out_hbm,
             head_v, rsp_v, esp_v, msk_v, remb_v, eemb_v, out_v, sem):
    cid = lax.axis_index("c")
    sid = lax.axis_index("s")
    wid = sid * 2 + cid
    base = wid * BPW

    pltpu.sync_copy(head_hbm.at[pl.ds(base, BPW)], head_v)
    c1 = pltpu.async_copy(rsp_hbm.at[head_v], rsp_v, sem)
    c2 = pltpu.async_copy(esp_hbm.at[head_v], esp_v, sem)
    c3 = pltpu.async_copy(mask_hbm.at[head_v], msk_v, sem)
    c1.wait()
    c2.wait()
    c3.wait()

    def head_body(i, carry):
        g0 = pltpu.async_copy(rel_hbm.at[rsp_v.at[i, pl.ds(0, HALF)]],
                              remb_v.at[pl.ds(0, HALF)], sem)
        g1 = pltpu.async_copy(rel_hbm.at[rsp_v.at[i, pl.ds(HALF, HALF)]],
                              remb_v.at[pl.ds(HALF, HALF)], sem)
        g2 = pltpu.async_copy(ent_hbm.at[esp_v.at[i, pl.ds(0, HALF)]],
                              eemb_v.at[pl.ds(0, HALF)], sem)
        g3 = pltpu.async_copy(ent_hbm.at[esp_v.at[i, pl.ds(HALF, HALF)]],
                              eemb_v.at[pl.ds(HALF, HALF)], sem)
        g0.wait()
        g1.wait()
        g2.wait()
        g3.wait()

        def grp_body(g, _):
            a0 = pl.multiple_of(g * 16, 16)
            mvec = msk_v[i, pl.ds(a0, 16)]
            for l in range(16):
                a = a0 + l
                mv = jnp.full((16,), mvec[l], dtype=jnp.float32)
                for c in range(4):
                    sl = pl.ds(c * 16, 16)
                    out_v[a, pl.ds(c * 16, 16)] = remb_v[a, sl] * mv
                    out_v[a, pl.ds(EMBED_DIM + c * 16, 16)] = eemb_v[a, sl] * mv
            return _

        lax.fori_loop(0, MAX_ACTIONS // 16, grp_body, None)
        pltpu.sync_copy(out_v, out_hbm.at[base + i])
        return carry

    lax.fori_loop(0, BPW, head_body, None)


@jax.jit
def _sc_call(ent_pad, rel_pad, action_mask, head, r_space, e_space):
    mesh = plsc.VectorSubcoreMesh(core_axis_name="c", subcore_axis_name="s")
    run = pl.kernel(
        _sc_body,
        out_type=jax.ShapeDtypeStruct((BATCH, MAX_ACTIONS, 2 * EMBED_DIM),
                                      jnp.float32),
        mesh=mesh,
        scratch_types=[
            pltpu.VMEM((BPW,), jnp.int32),
            pltpu.VMEM((BPW, MAX_ACTIONS), jnp.int32),
            pltpu.VMEM((BPW, MAX_ACTIONS), jnp.int32),
            pltpu.VMEM((BPW, MAX_ACTIONS), jnp.float32),
            pltpu.VMEM((MAX_ACTIONS, PAD_D), jnp.float32),
            pltpu.VMEM((MAX_ACTIONS, PAD_D), jnp.float32),
            pltpu.VMEM((MAX_ACTIONS, 2 * EMBED_DIM), jnp.float32),
            pltpu.SemaphoreType.DMA,
        ],
    )
    return run(ent_pad, rel_pad, action_mask, head, r_space, e_space)


def kernel(entity_table, relation_table, action_mask, head, r_space, e_space):
    head = head.astype(jnp.int32)
    ent_pad = jnp.pad(entity_table, ((0, 0), (0, PAD_D - EMBED_DIM)))
    rel_pad = jnp.pad(relation_table, ((0, 0), (0, PAD_D - EMBED_DIM)))
    return _sc_call(ent_pad, rel_pad, action_mask, head, r_space, e_space)
